# baseline (speedup 1.0000x reference)
import functools

import jax
import jax.numpy as jnp
from jax import lax
from jax.experimental import pallas as pl
from jax.experimental.pallas import tpu as pltpu
from jax.experimental.pallas import tpu_sc as plsc

_NC = 2
_NS = 16
_L = 16
_BS = 4096
_CHUNK = 16
_STRIP = 32
_SENT = 1 << 28


def _fm_body(F, FP, K, BPW, NBLK, cat_hbm, comb_hbm, bias_hbm, out_hbm,
             sp, acc, idx_v, clist, plist, out_v, bias_v,
             gb0, gb1, gsem0, gsem1, isem):
  KG = K // _L
  D = K + _L
  gbufs = (gb0, gb1)
  gsems = (gsem0, gsem1)

  cid = lax.axis_index("c")
  sid = lax.axis_index("s")
  wid = sid * _NC + cid
  base = wid * BPW
  zero = jnp.zeros((_L,), jnp.float32)
  izero = jnp.zeros((_L,), jnp.int32)
  lane_iota = lax.iota(jnp.int32, _L)

  pltpu.sync_copy(bias_hbm, bias_v)
  bias0 = bias_v[...][0]

  def _z(i, zc):
    acc[pl.ds(i * _L, _L)] = zero
    return zc

  lax.fori_loop(0, (BPW + 1) * D // _L, _z, 0)

  @pl.when(sid == 0)
  def _():
    for rr in range(_L):
      for gg in range(D // _L):
        gb0[rr, pl.ds(gg * _L, _L)] = zero
    pltpu.sync_copy(gb0.at[pl.ds(0, _L)], sp.at[pl.ds(_BS, _L)])

  def _idx_fire(c):
    pltpu.async_copy(
        cat_hbm.at[pl.ds(base + c * _CHUNK, _CHUNK)],
        idx_v.at[lax.rem(c, 2)], isem)

  def _idx_wait(c):
    pltpu.make_async_copy(
        cat_hbm.at[pl.ds(base + c * _CHUNK, _CHUNK)],
        idx_v.at[lax.rem(c, 2)], isem).wait()

  def _block(p, carry):
    _idx_fire(p * 0)
    plsc.subcore_barrier()
    rows = _BS // _NS
    pltpu.sync_copy(comb_hbm.at[pl.ds(p * _BS + sid * rows, rows)],
                    sp.at[pl.ds(sid * rows, rows)])
    plsc.subcore_barrier()

    def _chunk(c, carry):
      _idx_wait(c)

      @pl.when(c < BPW // _CHUNK - 1)
      def _():
        _idx_fire(c + 1)

      slot = lax.rem(c, 2)

      def _row(r, cpos):
        jloc = c * _CHUNK + r
        for g in range(FP // _L):
          v = idx_v[slot, r, pl.ds(g * _L, _L)]
          m = lax.shift_right_logical(v, 12) == p
          t = lax.bitwise_and(v, _BS - 1)
          plsc.store_compressed(clist.at[pl.ds(cpos, _L)], t, mask=m)
          plsc.store_compressed(plist.at[pl.ds(cpos, _L)], izero + jloc,
                                mask=m)
          cpos = cpos + plsc.all_reduce_population_count(m)[0]
        return cpos

      cnt = lax.fori_loop(0, _CHUNK, _row, jnp.int32(0))

      full = lane_iota >= 0
      for q in range(_STRIP // _L):
        plsc.store_compressed(clist.at[pl.ds(cnt + q * _L, _L)],
                              izero + _BS, mask=full)
        plsc.store_compressed(plist.at[pl.ds(cnt + q * _L, _L)],
                              izero + BPW, mask=full)

      nstrips = lax.div(cnt + (_STRIP - 1), _STRIP)

      def _gfire(s, b):
        pltpu.async_copy(sp.at[clist.at[pl.ds(s * _STRIP, _STRIP)]],
                         gbufs[b], gsems[b])

      def _gwait(s, b):
        pltpu.make_async_copy(sp.at[clist.at[pl.ds(s * _STRIP, _STRIP)]],
                              gbufs[b], gsems[b]).wait()

      @pl.when(nstrips > 0)
      def _():
        _gfire(jnp.int32(0), 0)

      @pl.when(nstrips > 1)
      def _():
        _gfire(jnp.int32(1), 1)

      def _pair(pr, pcarry):
        for b in range(2):
          s = pr * 2 + b

          @pl.when(s < nstrips)
          def _():
            _gwait(s, b)
            for i16 in range(_STRIP // _L):
              pv = plist[pl.ds(s * _STRIP + i16 * _L, _L)]
              for i in range(_L):
                row = i16 * _L + i
                off = pv[i] * D
                for gg in range(D // _L):
                  plsc.addupdate(acc.at[pl.ds(off + gg * _L, _L)],
                                 gbufs[b][row, pl.ds(gg * _L, _L)])

            @pl.when(s + 2 < nstrips)
            def _():
              _gfire(s + 2, b)

        return pcarry

      lax.fori_loop(0, lax.div(nstrips + 1, 2), _pair, 0)
      return carry

    lax.fori_loop(0, BPW // _CHUNK, _chunk, 0)
    return carry

  lax.fori_loop(0, NBLK, _block, 0)

  def _ep(j, resvec):
    off = j * D
    d = zero
    for gg in range(KG):
      s = acc[pl.ds(off + gg * _L, _L)]
      d = d + s * s
    v = 0.5 * d + acc[pl.ds(off + KG * _L, _L)]
    total = bias0
    for lane_i in range(_L):
      total = total + v[lane_i]
    lane = lax.bitwise_and(j, _L - 1)
    resvec = jnp.where(lane_iota == lane, total, resvec)

    @pl.when(lane == _L - 1)
    def _():
      out_v[pl.ds(j - (_L - 1), _L)] = resvec

    return resvec

  lax.fori_loop(0, BPW, _ep, zero)
  pltpu.sync_copy(out_v, out_hbm.at[pl.ds(base, BPW)])


def _aux_tc(emb_pad, lin3, VP, K):
  RB = 1024
  D = K + _L

  def body(emb_ref, lin_ref, out_ref):
    e = emb_ref[...]
    aux = lin_ref[0, 0] - 0.5 * jnp.sum(e * e, axis=1)
    out_ref[...] = jnp.concatenate(
        [e, aux[:, None], jnp.zeros((RB, _L - 1), jnp.float32)], axis=1)

  return pl.pallas_call(
      body,
      grid=(VP // RB,),
      in_specs=[
          pl.BlockSpec((RB, K), lambda i: (i, 0)),
          pl.BlockSpec((1, 1, RB), lambda i: (i, 0, 0)),
      ],
      out_specs=pl.BlockSpec((RB, D), lambda i: (i, 0)),
      out_shape=jax.ShapeDtypeStruct((VP, D), jnp.float32),
  )(emb_pad, lin3)


def kernel(cat_features, emb_table, lin_table, bias):
  B, F = cat_features.shape
  V, K = emb_table.shape
  NW = _NC * _NS
  BPW = B // NW
  FP = -(-F // _L) * _L
  NBLK = -(-V // _BS)
  VP = NBLK * _BS
  D = K + _L

  cat_pad = jnp.pad(cat_features, ((0, 0), (0, FP - F)),
                    constant_values=1 << 28)
  emb_pad = jnp.pad(emb_table, ((0, VP - V), (0, 0)))
  lin3 = jnp.pad(lin_table[:, 0], (0, VP - V)).reshape(VP // 1024, 1, 1024)
  bias_pad = jnp.pad(bias, (0, _L - bias.shape[0]))

  comb = _aux_tc(emb_pad, lin3, VP, K)

  mesh = plsc.VectorSubcoreMesh(core_axis_name="c", subcore_axis_name="s")
  nent = _CHUNK * FP + 2 * _STRIP
  scratch = [
      pltpu.VMEM_SHARED((_BS + _L, D), jnp.float32),
      pltpu.VMEM(((BPW + 1) * D,), jnp.float32),
      pltpu.VMEM((2, _CHUNK, FP), jnp.int32),
      pltpu.VMEM((nent,), jnp.int32),
      pltpu.VMEM((nent,), jnp.int32),
      pltpu.VMEM((BPW,), jnp.float32),
      pltpu.VMEM((_L,), jnp.float32),
      pltpu.VMEM((_STRIP, D), jnp.float32),
      pltpu.VMEM((_STRIP, D), jnp.float32),
      pltpu.SemaphoreType.DMA,
      pltpu.SemaphoreType.DMA,
      pltpu.SemaphoreType.DMA,
  ]

  body = functools.partial(_fm_body, F, FP, K, BPW, NBLK)
  out = pl.kernel(
      body,
      out_type=jax.ShapeDtypeStruct((B,), jnp.float32),
      mesh=mesh,
      scratch_types=scratch,
      compiler_params=pltpu.CompilerParams(
          use_tc_tiling_on_sc=False, needs_layout_passes=False),
  )(cat_pad, comb, bias_pad)
  return out.reshape(B, 1)
